# baseline (speedup 1.0000x reference)
import functools

import jax
import jax.numpy as jnp
from jax import lax
from jax.experimental import pallas as pl
from jax.experimental.pallas import tpu as pltpu
from jax.experimental.pallas import tpu_sc as plsc

BATCH = 16384
FACTOR = 64
NUM_WORKERS = 32
ROWS_PER_WORKER = BATCH // NUM_WORKERS
BLOCKS = ROWS_PER_WORKER // 16


def _clmf_body(user_hbm, item_hbm, ut_hbm, it_hbm, wb_hbm, out_hbm,
               idx_u, idx_i, eu, ei, out_v, wb_v, sem_u, sem_i):
    wid = lax.axis_index("s") * 2 + lax.axis_index("c")
    base = wid * ROWS_PER_WORKER

    pltpu.sync_copy(user_hbm.at[pl.ds(base, ROWS_PER_WORKER)], idx_u)
    pltpu.sync_copy(item_hbm.at[pl.ds(base, ROWS_PER_WORKER)], idx_i)
    pltpu.sync_copy(wb_hbm, wb_v)

    cu = pltpu.async_copy(ut_hbm.at[idx_u], eu, sem_u)
    ci = pltpu.async_copy(it_hbm.at[idx_i], ei, sem_i)
    cu.wait()
    ci.wait()

    lane = lax.iota(jnp.int32, 16)
    w_vecs = [wb_v[pl.ds(16 * q, 16)] for q in range(FACTOR // 16)]
    bias_vec = wb_v[pl.ds(FACTOR, 16)]

    def block(blk, carry):
        rows = blk * 16 + lane
        acc = jnp.zeros((16,), jnp.float32)
        for f in range(FACTOR):
            cols = jnp.full((16,), f, jnp.int32)
            eu_f = plsc.load_gather(eu, [rows, cols])
            ei_f = plsc.load_gather(ei, [rows, cols])
            acc = acc + eu_f * ei_f * w_vecs[f // 16][f % 16]
        out_v[pl.ds(blk * 16, 16)] = acc + bias_vec
        return carry

    lax.fori_loop(0, BLOCKS, block, 0)

    pltpu.sync_copy(out_v, out_hbm.at[pl.ds(base, ROWS_PER_WORKER)])


@jax.jit
def _clmf_call(user, item, embed_user_w, embed_item_w, wb):
    mesh = plsc.VectorSubcoreMesh(core_axis_name="c", subcore_axis_name="s")
    kern = pl.kernel(
        _clmf_body,
        out_type=jax.ShapeDtypeStruct((BATCH,), jnp.float32),
        mesh=mesh,
        compiler_params=pltpu.CompilerParams(
            needs_layout_passes=False, use_tc_tiling_on_sc=False),
        scratch_types=[
            pltpu.VMEM((ROWS_PER_WORKER,), jnp.int32),
            pltpu.VMEM((ROWS_PER_WORKER,), jnp.int32),
            pltpu.VMEM((ROWS_PER_WORKER, FACTOR), jnp.float32),
            pltpu.VMEM((ROWS_PER_WORKER, FACTOR), jnp.float32),
            pltpu.VMEM((ROWS_PER_WORKER,), jnp.float32),
            pltpu.VMEM((FACTOR + 16,), jnp.float32),
            pltpu.SemaphoreType.DMA,
            pltpu.SemaphoreType.DMA,
        ],
    )
    return kern(user, item, embed_user_w, embed_item_w, wb)


def kernel(user, item, embed_user_w, embed_item_w, predict_w, predict_b):
    user = user.astype(jnp.int32)
    item = item.astype(jnp.int32)
    w = predict_w.reshape(FACTOR).astype(jnp.float32)
    b = jnp.broadcast_to(predict_b.astype(jnp.float32), (16,))
    wb = jnp.concatenate([w, b])
    return _clmf_call(user, item, embed_user_w, embed_item_w, wb)
